# Initial kernel scaffold; baseline (speedup 1.0000x reference)
import jax
import jax.numpy as jnp
from jax.experimental import pallas as pl


def kernel(inputs):
    raise NotImplementedError("write your pallas kernel here")


# speedup vs baseline: 1.2296x; 1.2296x over previous
import functools

import jax
import jax.numpy as jnp
from jax.experimental import pallas as pl
from jax.experimental.pallas import tpu as pltpu

_T = 5
_G = 4
_EPS = 1e-5


def _cov_kernel(z_ref, s_ref, cs_ref, acc_ref, cs_acc_ref, *, n_chunks):
    c = pl.program_id(1)

    @pl.when(c == 0)
    def _():
        acc_ref[...] = jnp.zeros_like(acc_ref)
        cs_acc_ref[...] = jnp.zeros_like(cs_acc_ref)

    zb = z_ref[0]
    acc_ref[...] += jax.lax.dot_general(
        zb, zb, (((0,), (0,)), ((), ())),
        preferred_element_type=jnp.float32)
    cs_acc_ref[...] += jnp.sum(zb, axis=0, keepdims=True)

    @pl.when(c == n_chunks - 1)
    def _():
        s_ref[0] = acc_ref[...]
        cs_ref[0] = cs_acc_ref[...]


def _ns_kernel(s_ref, cs_ref, m_ref, mu_ref, *, n_samples):
    d = s_ref.shape[-1]
    s_raw = s_ref[0]
    mu = cs_ref[0] * (1.0 / n_samples)
    outer = jax.lax.dot_general(
        mu, mu, (((0,), (0,)), ((), ())),
        preferred_element_type=jnp.float32)
    rows = jax.lax.broadcasted_iota(jnp.int32, (d, d), 0)
    cols = jax.lax.broadcasted_iota(jnp.int32, (d, d), 1)
    eye = jnp.where(rows == cols, jnp.float32(1.0), jnp.float32(0.0))
    s = s_raw - n_samples * outer + _EPS * eye
    norm = jnp.sqrt(jnp.sum(s * s))
    s = s * (1.0 / norm)
    b = eye
    for _ in range(_T):
        b3 = jnp.dot(jnp.dot(b, b, preferred_element_type=jnp.float32), b,
                     preferred_element_type=jnp.float32)
        b = 1.5 * b - 0.5 * jnp.dot(b3, s, preferred_element_type=jnp.float32)
    m_ref[0] = b.T * jax.lax.rsqrt(norm)
    mu_ref[0] = mu


def _apply_kernel(z_ref, m_ref, mu_ref, o_ref):
    zb = z_ref[0] - mu_ref[0]
    o_ref[0] = jnp.dot(zb, m_ref[0], preferred_element_type=jnp.float32)


def kernel(inputs):
    k = inputs.shape
    c = k[-1]
    g = _G
    d = c // g
    z = inputs.reshape(g, -1, d)
    n = z.shape[1]

    nb = 2048
    n_chunks = n // nb

    s_raw, cs = pl.pallas_call(
        functools.partial(_cov_kernel, n_chunks=n_chunks),
        grid=(g, n_chunks),
        in_specs=[pl.BlockSpec((1, nb, d), lambda i, j: (i, j, 0))],
        out_specs=[
            pl.BlockSpec((1, d, d), lambda i, j: (i, 0, 0)),
            pl.BlockSpec((1, 1, d), lambda i, j: (i, 0, 0)),
        ],
        out_shape=[
            jax.ShapeDtypeStruct((g, d, d), jnp.float32),
            jax.ShapeDtypeStruct((g, 1, d), jnp.float32),
        ],
        scratch_shapes=[
            pltpu.VMEM((d, d), jnp.float32),
            pltpu.VMEM((1, d), jnp.float32),
        ],
        compiler_params=pltpu.CompilerParams(
            dimension_semantics=("parallel", "arbitrary")),
        name="oni_cov",
    )(z)

    m, mu = pl.pallas_call(
        functools.partial(_ns_kernel, n_samples=n),
        grid=(g,),
        in_specs=[
            pl.BlockSpec((1, d, d), lambda i: (i, 0, 0)),
            pl.BlockSpec((1, 1, d), lambda i: (i, 0, 0)),
        ],
        out_specs=[
            pl.BlockSpec((1, d, d), lambda i: (i, 0, 0)),
            pl.BlockSpec((1, 1, d), lambda i: (i, 0, 0)),
        ],
        out_shape=[
            jax.ShapeDtypeStruct((g, d, d), jnp.float32),
            jax.ShapeDtypeStruct((g, 1, d), jnp.float32),
        ],
        compiler_params=pltpu.CompilerParams(
            dimension_semantics=("parallel",)),
        name="oni_ns",
    )(s_raw, cs)

    out = pl.pallas_call(
        _apply_kernel,
        grid=(g, n_chunks),
        in_specs=[
            pl.BlockSpec((1, nb, d), lambda i, j: (i, j, 0)),
            pl.BlockSpec((1, d, d), lambda i, j: (i, 0, 0)),
            pl.BlockSpec((1, 1, d), lambda i, j: (i, 0, 0)),
        ],
        out_specs=pl.BlockSpec((1, nb, d), lambda i, j: (i, j, 0)),
        out_shape=jax.ShapeDtypeStruct(z.shape, jnp.float32),
        compiler_params=pltpu.CompilerParams(
            dimension_semantics=("parallel", "arbitrary")),
        name="oni_apply",
    )(z, m, mu)

    return out.reshape(k)


# speedup vs baseline: 3.5382x; 2.8776x over previous
import functools

import jax
import jax.numpy as jnp
from jax.experimental import pallas as pl
from jax.experimental.pallas import tpu as pltpu

_T = 5
_G = 4
_EPS = 1e-5


def _cov_kernel(x_ref, s_ref, cs_ref, acc_ref, cs_acc_ref, *, n_chunks, d):
    c = pl.program_id(1)

    @pl.when(c == 0)
    def _():
        acc_ref[...] = jnp.zeros_like(acc_ref)
        cs_acc_ref[...] = jnp.zeros_like(cs_acc_ref)

    xb = x_ref[...]
    n_panels = xb.shape[1] // d
    acc = acc_ref[...]
    cs = cs_acc_ref[...]
    for k in range(n_panels):
        panel = xb[:, k * d:(k + 1) * d]
        acc += jax.lax.dot_general(
            panel, panel, (((0,), (0,)), ((), ())),
            preferred_element_type=jnp.float32)
        cs += jnp.sum(panel, axis=0, keepdims=True)
    acc_ref[...] = acc
    cs_acc_ref[...] = cs

    @pl.when(c == n_chunks - 1)
    def _():
        s_ref[0] = acc_ref[...]
        cs_ref[0] = cs_acc_ref[...]


def _ns_kernel(s_ref, cs_ref, m_ref, mu_ref, *, n_samples):
    d = s_ref.shape[-1]
    s_raw = s_ref[0]
    mu = cs_ref[0] * (1.0 / n_samples)
    outer = jax.lax.dot_general(
        mu, mu, (((0,), (0,)), ((), ())),
        preferred_element_type=jnp.float32)
    rows = jax.lax.broadcasted_iota(jnp.int32, (d, d), 0)
    cols = jax.lax.broadcasted_iota(jnp.int32, (d, d), 1)
    eye = jnp.where(rows == cols, jnp.float32(1.0), jnp.float32(0.0))
    s = s_raw - n_samples * outer + _EPS * eye
    norm = jnp.sqrt(jnp.sum(s * s))
    s = s * (1.0 / norm)
    b = eye
    for _ in range(_T):
        b3 = jnp.dot(jnp.dot(b, b, preferred_element_type=jnp.float32), b,
                     preferred_element_type=jnp.float32)
        b = 1.5 * b - 0.5 * jnp.dot(b3, s, preferred_element_type=jnp.float32)
    m_ref[0] = b.T * jax.lax.rsqrt(norm)
    mu_ref[0] = mu


def _apply_kernel(x_ref, m_ref, mu_ref, o_ref, *, d):
    xb = x_ref[...]
    n_panels = xb.shape[1] // d
    m = m_ref[0]
    mu = mu_ref[0]
    for k in range(n_panels):
        panel = xb[:, k * d:(k + 1) * d] - mu
        o_ref[:, k * d:(k + 1) * d] = jnp.dot(
            panel, m, preferred_element_type=jnp.float32)


def kernel(inputs):
    rows, c = inputs.shape
    g = _G
    d = c // g
    band = rows // g
    n_samples = band * (c // d)

    r = 1024
    n_chunks = band // r

    s_raw, cs = pl.pallas_call(
        functools.partial(_cov_kernel, n_chunks=n_chunks, d=d),
        grid=(g, n_chunks),
        in_specs=[pl.BlockSpec((r, c), lambda i, j: (i * n_chunks + j, 0))],
        out_specs=[
            pl.BlockSpec((1, d, d), lambda i, j: (i, 0, 0)),
            pl.BlockSpec((1, 1, d), lambda i, j: (i, 0, 0)),
        ],
        out_shape=[
            jax.ShapeDtypeStruct((g, d, d), jnp.float32),
            jax.ShapeDtypeStruct((g, 1, d), jnp.float32),
        ],
        scratch_shapes=[
            pltpu.VMEM((d, d), jnp.float32),
            pltpu.VMEM((1, d), jnp.float32),
        ],
        compiler_params=pltpu.CompilerParams(
            dimension_semantics=("parallel", "arbitrary"),
            vmem_limit_bytes=56 * 1024 * 1024),
        name="oni_cov",
    )(inputs)

    m, mu = pl.pallas_call(
        functools.partial(_ns_kernel, n_samples=n_samples),
        grid=(g,),
        in_specs=[
            pl.BlockSpec((1, d, d), lambda i: (i, 0, 0)),
            pl.BlockSpec((1, 1, d), lambda i: (i, 0, 0)),
        ],
        out_specs=[
            pl.BlockSpec((1, d, d), lambda i: (i, 0, 0)),
            pl.BlockSpec((1, 1, d), lambda i: (i, 0, 0)),
        ],
        out_shape=[
            jax.ShapeDtypeStruct((g, d, d), jnp.float32),
            jax.ShapeDtypeStruct((g, 1, d), jnp.float32),
        ],
        compiler_params=pltpu.CompilerParams(
            dimension_semantics=("parallel",)),
        name="oni_ns",
    )(s_raw, cs)

    out = pl.pallas_call(
        functools.partial(_apply_kernel, d=d),
        grid=(g, n_chunks),
        in_specs=[
            pl.BlockSpec((r, c), lambda i, j: (i * n_chunks + j, 0)),
            pl.BlockSpec((1, d, d), lambda i, j: (i, 0, 0)),
            pl.BlockSpec((1, 1, d), lambda i, j: (i, 0, 0)),
        ],
        out_specs=pl.BlockSpec((r, c), lambda i, j: (i * n_chunks + j, 0)),
        out_shape=jax.ShapeDtypeStruct(inputs.shape, jnp.float32),
        compiler_params=pltpu.CompilerParams(
            dimension_semantics=("parallel", "arbitrary"),
            vmem_limit_bytes=56 * 1024 * 1024),
        name="oni_apply",
    )(inputs, m, mu)

    return out


# speedup vs baseline: 3.6382x; 1.0283x over previous
import functools

import jax
import jax.numpy as jnp
from jax.experimental import pallas as pl
from jax.experimental.pallas import tpu as pltpu

_T = 5
_G = 4
_EPS = 1e-5


def _cov_ns_kernel(x_ref, m_ref, mu_ref, acc_ref, cs_acc_ref,
                   *, n_chunks, d, n_samples):
    c = pl.program_id(1)

    @pl.when(c == 0)
    def _():
        acc_ref[...] = jnp.zeros_like(acc_ref)
        cs_acc_ref[...] = jnp.zeros_like(cs_acc_ref)

    xb = x_ref[...]
    n_panels = xb.shape[1] // d
    acc = acc_ref[...]
    cs = cs_acc_ref[...]
    for k in range(n_panels):
        panel = xb[:, k * d:(k + 1) * d]
        acc += jax.lax.dot_general(
            panel, panel, (((0,), (0,)), ((), ())),
            preferred_element_type=jnp.float32)
        cs += jnp.sum(panel, axis=0, keepdims=True)
    acc_ref[...] = acc
    cs_acc_ref[...] = cs

    @pl.when(c == n_chunks - 1)
    def _():
        s_raw = acc_ref[...]
        mu = cs_acc_ref[...] * (1.0 / n_samples)
        outer = jax.lax.dot_general(
            mu, mu, (((0,), (0,)), ((), ())),
            preferred_element_type=jnp.float32)
        rows = jax.lax.broadcasted_iota(jnp.int32, (d, d), 0)
        cols = jax.lax.broadcasted_iota(jnp.int32, (d, d), 1)
        eye = jnp.where(rows == cols, jnp.float32(1.0), jnp.float32(0.0))
        s = s_raw - n_samples * outer + _EPS * eye
        norm = jnp.sqrt(jnp.sum(s * s))
        s = s * (1.0 / norm)
        b = eye
        for _ in range(_T):
            b3 = jnp.dot(jnp.dot(b, b, preferred_element_type=jnp.float32), b,
                         preferred_element_type=jnp.float32)
            b = 1.5 * b - 0.5 * jnp.dot(b3, s,
                                        preferred_element_type=jnp.float32)
        m_ref[0] = b.T * jax.lax.rsqrt(norm)
        mu_ref[0] = mu


def _apply_kernel(x_ref, m_ref, mu_ref, o_ref, *, d):
    xb = x_ref[...]
    n_panels = xb.shape[1] // d
    m = m_ref[0]
    mu = mu_ref[0]
    for k in range(n_panels):
        panel = xb[:, k * d:(k + 1) * d] - mu
        o_ref[:, k * d:(k + 1) * d] = jnp.dot(
            panel, m, preferred_element_type=jnp.float32)


def kernel(inputs):
    rows, c = inputs.shape
    g = _G
    d = c // g
    band = rows // g
    n_samples = band * (c // d)

    r_cov = 2048
    r_app = 1024
    nc_cov = band // r_cov
    nc_app = band // r_app

    m, mu = pl.pallas_call(
        functools.partial(_cov_ns_kernel, n_chunks=nc_cov, d=d,
                          n_samples=n_samples),
        grid=(g, nc_cov),
        in_specs=[pl.BlockSpec((r_cov, c), lambda i, j: (i * nc_cov + j, 0))],
        out_specs=[
            pl.BlockSpec((1, d, d), lambda i, j: (i, 0, 0)),
            pl.BlockSpec((1, 1, d), lambda i, j: (i, 0, 0)),
        ],
        out_shape=[
            jax.ShapeDtypeStruct((g, d, d), jnp.float32),
            jax.ShapeDtypeStruct((g, 1, d), jnp.float32),
        ],
        scratch_shapes=[
            pltpu.VMEM((d, d), jnp.float32),
            pltpu.VMEM((1, d), jnp.float32),
        ],
        compiler_params=pltpu.CompilerParams(
            dimension_semantics=("parallel", "arbitrary"),
            vmem_limit_bytes=56 * 1024 * 1024),
        name="oni_cov_ns",
    )(inputs)

    out = pl.pallas_call(
        functools.partial(_apply_kernel, d=d),
        grid=(g, nc_app),
        in_specs=[
            pl.BlockSpec((r_app, c), lambda i, j: (i * nc_app + j, 0)),
            pl.BlockSpec((1, d, d), lambda i, j: (i, 0, 0)),
            pl.BlockSpec((1, 1, d), lambda i, j: (i, 0, 0)),
        ],
        out_specs=pl.BlockSpec((r_app, c), lambda i, j: (i * nc_app + j, 0)),
        out_shape=jax.ShapeDtypeStruct(inputs.shape, jnp.float32),
        compiler_params=pltpu.CompilerParams(
            dimension_semantics=("parallel", "arbitrary"),
            vmem_limit_bytes=56 * 1024 * 1024),
        name="oni_apply",
    )(inputs, m, mu)

    return out


# speedup vs baseline: 4.0188x; 1.1046x over previous
import functools

import jax
import jax.numpy as jnp
from jax.experimental import pallas as pl
from jax.experimental.pallas import tpu as pltpu

_T = 5
_G = 4
_EPS = 1e-5


def _oni_kernel(x_ref, o_ref, cache_ref, acc_ref, cs_ref, m_ref, mu_ref,
                *, nc, d, n_samples):
    j = pl.program_id(1)

    @pl.when(j == 0)
    def _():
        acc_ref[...] = jnp.zeros_like(acc_ref)
        cs_ref[...] = jnp.zeros_like(cs_ref)

    @pl.when(j < nc)
    def _cov_phase():
        xb = x_ref[...]
        cache_ref[j] = xb
        n_panels = xb.shape[1] // d
        acc = acc_ref[...]
        cs = cs_ref[...]
        for k in range(n_panels):
            panel = xb[:, k * d:(k + 1) * d]
            acc += jax.lax.dot_general(
                panel, panel, (((0,), (0,)), ((), ())),
                preferred_element_type=jnp.float32)
            cs += jnp.sum(panel, axis=0, keepdims=True)
        acc_ref[...] = acc
        cs_ref[...] = cs

        @pl.when(j == nc - 1)
        def _ns_epilogue():
            s_raw = acc_ref[...]
            mu = cs_ref[...] * (1.0 / n_samples)
            outer = jax.lax.dot_general(
                mu, mu, (((0,), (0,)), ((), ())),
                preferred_element_type=jnp.float32)
            rows = jax.lax.broadcasted_iota(jnp.int32, (d, d), 0)
            cols = jax.lax.broadcasted_iota(jnp.int32, (d, d), 1)
            eye = jnp.where(rows == cols, jnp.float32(1.0), jnp.float32(0.0))
            s = s_raw - n_samples * outer + _EPS * eye
            norm = jnp.sqrt(jnp.sum(s * s))
            s = s * (1.0 / norm)
            b = eye
            for _ in range(_T):
                b3 = jnp.dot(
                    jnp.dot(b, b, preferred_element_type=jnp.float32), b,
                    preferred_element_type=jnp.float32)
                b = 1.5 * b - 0.5 * jnp.dot(
                    b3, s, preferred_element_type=jnp.float32)
            m_ref[...] = b.T * jax.lax.rsqrt(norm)
            mu_ref[...] = mu

    @pl.when(j >= nc)
    def _apply_phase():
        jj = j - nc
        xb = cache_ref[jj]
        n_panels = xb.shape[1] // d
        m = m_ref[...]
        mu = mu_ref[...]
        for k in range(n_panels):
            panel = xb[:, k * d:(k + 1) * d] - mu
            o_ref[:, k * d:(k + 1) * d] = jnp.dot(
                panel, m, preferred_element_type=jnp.float32)


def kernel(inputs):
    rows, c = inputs.shape
    g = _G
    d = c // g
    band = rows // g
    n_samples = band * (c // d)

    r = 512
    nc = band // r

    out = pl.pallas_call(
        functools.partial(_oni_kernel, nc=nc, d=d, n_samples=n_samples),
        grid=(g, 2 * nc),
        in_specs=[
            pl.BlockSpec(
                (r, c), lambda i, j: (i * nc + jnp.minimum(j, nc - 1), 0)),
        ],
        out_specs=pl.BlockSpec(
            (r, c), lambda i, j: (i * nc + jnp.maximum(j - nc, 0), 0)),
        out_shape=jax.ShapeDtypeStruct(inputs.shape, jnp.float32),
        scratch_shapes=[
            pltpu.VMEM((nc, r, c), jnp.float32),
            pltpu.VMEM((d, d), jnp.float32),
            pltpu.VMEM((1, d), jnp.float32),
            pltpu.VMEM((d, d), jnp.float32),
            pltpu.VMEM((1, d), jnp.float32),
        ],
        compiler_params=pltpu.CompilerParams(
            dimension_semantics=("parallel", "arbitrary"),
            vmem_limit_bytes=60 * 1024 * 1024),
        name="oni_fused",
    )(inputs)

    return out


# speedup vs baseline: 4.2795x; 1.0649x over previous
import functools

import jax
import jax.numpy as jnp
from jax.experimental import pallas as pl
from jax.experimental.pallas import tpu as pltpu

_T = 5
_G = 4
_EPS = 1e-5


def _oni_kernel(x_ref, o_ref, cache_ref, acc_ref, cs_ref, m_ref, off_ref,
                *, nc, d, n_samples):
    j = pl.program_id(1)

    @pl.when(j == 0)
    def _():
        acc_ref[...] = jnp.zeros_like(acc_ref)
        cs_ref[...] = jnp.zeros_like(cs_ref)

    @pl.when(j < nc)
    def _cov_phase():
        xb = x_ref[...].astype(jnp.bfloat16)
        cache_ref[j] = xb
        n_panels = xb.shape[1] // d
        acc = acc_ref[...]
        cs = cs_ref[...]
        for k in range(n_panels):
            panel = xb[:, k * d:(k + 1) * d]
            acc += jax.lax.dot_general(
                panel, panel, (((0,), (0,)), ((), ())),
                preferred_element_type=jnp.float32)
            cs += jnp.sum(panel.astype(jnp.float32), axis=0, keepdims=True)
        acc_ref[...] = acc
        cs_ref[...] = cs

        @pl.when(j == nc - 1)
        def _ns_epilogue():
            s_raw = acc_ref[...]
            mu = cs_ref[...] * (1.0 / n_samples)
            outer = jax.lax.dot_general(
                mu, mu, (((0,), (0,)), ((), ())),
                preferred_element_type=jnp.float32)
            rows = jax.lax.broadcasted_iota(jnp.int32, (d, d), 0)
            cols = jax.lax.broadcasted_iota(jnp.int32, (d, d), 1)
            eye = jnp.where(rows == cols, jnp.float32(1.0), jnp.float32(0.0))
            s = s_raw - n_samples * outer + _EPS * eye
            norm = jnp.sqrt(jnp.sum(s * s))
            s = s * (1.0 / norm)
            b = eye
            for _ in range(_T):
                b3 = jnp.dot(
                    jnp.dot(b, b, preferred_element_type=jnp.float32), b,
                    preferred_element_type=jnp.float32)
                b = 1.5 * b - 0.5 * jnp.dot(
                    b3, s, preferred_element_type=jnp.float32)
            m = b.T * jax.lax.rsqrt(norm)
            m_ref[...] = m.astype(jnp.bfloat16)
            off_ref[...] = jnp.dot(mu, m, preferred_element_type=jnp.float32)

    @pl.when(j >= nc)
    def _apply_phase():
        jj = j - nc
        xb = cache_ref[jj]
        n_panels = xb.shape[1] // d
        m = m_ref[...]
        off = off_ref[...]
        for k in range(n_panels):
            panel = xb[:, k * d:(k + 1) * d]
            o_ref[:, k * d:(k + 1) * d] = jnp.dot(
                panel, m, preferred_element_type=jnp.float32) - off


def kernel(inputs):
    rows, c = inputs.shape
    g = _G
    d = c // g
    band = rows // g
    n_samples = band * (c // d)

    r = 1024
    nc = band // r

    out = pl.pallas_call(
        functools.partial(_oni_kernel, nc=nc, d=d, n_samples=n_samples),
        grid=(g, 2 * nc),
        in_specs=[
            pl.BlockSpec(
                (r, c), lambda i, j: (i * nc + jnp.minimum(j, nc - 1), 0)),
        ],
        out_specs=pl.BlockSpec(
            (r, c), lambda i, j: (i * nc + jnp.maximum(j - nc, 0), 0)),
        out_shape=jax.ShapeDtypeStruct(inputs.shape, jnp.float32),
        scratch_shapes=[
            pltpu.VMEM((nc, r, c), jnp.bfloat16),
            pltpu.VMEM((d, d), jnp.float32),
            pltpu.VMEM((1, d), jnp.float32),
            pltpu.VMEM((d, d), jnp.bfloat16),
            pltpu.VMEM((1, d), jnp.float32),
        ],
        compiler_params=pltpu.CompilerParams(
            dimension_semantics=("parallel", "arbitrary"),
            vmem_limit_bytes=60 * 1024 * 1024),
        name="oni_fused",
    )(inputs)

    return out


# speedup vs baseline: 4.3150x; 1.0083x over previous
import functools

import jax
import jax.numpy as jnp
from jax.experimental import pallas as pl
from jax.experimental.pallas import tpu as pltpu

_T = 5
_G = 4
_EPS = 1e-5


def _oni_kernel(x_ref, o_ref, cache_ref, acc_ref, cs_ref, m_ref, off_ref,
                *, nc, d, n_samples):
    j = pl.program_id(1)

    @pl.when(j == 0)
    def _():
        acc_ref[...] = jnp.zeros_like(acc_ref)
        cs_ref[...] = jnp.zeros_like(cs_ref)

    @pl.when(j < nc)
    def _cov_phase():
        xb = x_ref[...].astype(jnp.bfloat16)
        cache_ref[j] = xb
        n_panels = xb.shape[1] // d
        acc = acc_ref[...]
        cs = cs_ref[...]
        for k in range(n_panels):
            panel = xb[:, k * d:(k + 1) * d]
            acc += jax.lax.dot_general(
                panel, panel, (((0,), (0,)), ((), ())),
                preferred_element_type=jnp.float32)
            cs += jnp.sum(panel.astype(jnp.float32), axis=0, keepdims=True)
        acc_ref[...] = acc
        cs_ref[...] = cs

        @pl.when(j == nc - 1)
        def _ns_epilogue():
            s_raw = acc_ref[...]
            mu = cs_ref[...] * (1.0 / n_samples)
            outer = jax.lax.dot_general(
                mu, mu, (((0,), (0,)), ((), ())),
                preferred_element_type=jnp.float32)
            rows = jax.lax.broadcasted_iota(jnp.int32, (d, d), 0)
            cols = jax.lax.broadcasted_iota(jnp.int32, (d, d), 1)
            eye = jnp.where(rows == cols, jnp.float32(1.0), jnp.float32(0.0))
            s = s_raw - n_samples * outer + _EPS * eye
            norm = jnp.sqrt(jnp.sum(s * s))
            s = s * (1.0 / norm)
            b = 1.5 * eye - 0.5 * s
            s_b = s.astype(jnp.bfloat16)
            for _ in range(_T - 1):
                b_b = b.astype(jnp.bfloat16)
                b2 = jnp.dot(b_b, b_b, preferred_element_type=jnp.float32)
                b3 = jnp.dot(b2.astype(jnp.bfloat16), b_b,
                             preferred_element_type=jnp.float32)
                b = 1.5 * b - 0.5 * jnp.dot(
                    b3.astype(jnp.bfloat16), s_b,
                    preferred_element_type=jnp.float32)
            m = b.T * jax.lax.rsqrt(norm)
            m_ref[...] = m.astype(jnp.bfloat16)
            off_ref[...] = jnp.dot(mu, m, preferred_element_type=jnp.float32)

    @pl.when(j >= nc)
    def _apply_phase():
        jj = j - nc
        xb = cache_ref[jj]
        n_panels = xb.shape[1] // d
        m = m_ref[...]
        off = off_ref[...]
        for k in range(n_panels):
            panel = xb[:, k * d:(k + 1) * d]
            o_ref[:, k * d:(k + 1) * d] = jnp.dot(
                panel, m, preferred_element_type=jnp.float32) - off


def kernel(inputs):
    rows, c = inputs.shape
    g = _G
    d = c // g
    band = rows // g
    n_samples = band * (c // d)

    r = 1024
    nc = band // r

    out = pl.pallas_call(
        functools.partial(_oni_kernel, nc=nc, d=d, n_samples=n_samples),
        grid=(g, 2 * nc),
        in_specs=[
            pl.BlockSpec(
                (r, c), lambda i, j: (i * nc + jnp.minimum(j, nc - 1), 0)),
        ],
        out_specs=pl.BlockSpec(
            (r, c), lambda i, j: (i * nc + jnp.maximum(j - nc, 0), 0)),
        out_shape=jax.ShapeDtypeStruct(inputs.shape, jnp.float32),
        scratch_shapes=[
            pltpu.VMEM((nc, r, c), jnp.bfloat16),
            pltpu.VMEM((d, d), jnp.float32),
            pltpu.VMEM((1, d), jnp.float32),
            pltpu.VMEM((d, d), jnp.bfloat16),
            pltpu.VMEM((1, d), jnp.float32),
        ],
        compiler_params=pltpu.CompilerParams(
            dimension_semantics=("parallel", "arbitrary"),
            vmem_limit_bytes=60 * 1024 * 1024),
        name="oni_fused",
    )(inputs)

    return out


# speedup vs baseline: 4.4342x; 1.0276x over previous
import functools

import jax
import jax.numpy as jnp
from jax.experimental import pallas as pl
from jax.experimental.pallas import tpu as pltpu

_T = 5
_G = 4
_EPS = 1e-5


def _oni_kernel(x_ref, o_ref, cache_ref, acc_ref, cs_ref, m_ref, off_ref,
                *, g, nc, d, n_samples):
    i = pl.program_id(0)
    j = pl.program_id(1)
    p = jax.lax.rem(i, 2)

    @pl.when(jnp.logical_and(i < g, j == 0))
    def _():
        acc_ref[...] = jnp.zeros_like(acc_ref)
        cs_ref[...] = jnp.zeros_like(cs_ref)

    @pl.when(i < g)
    def _cov_phase():
        xb = x_ref[...].astype(jnp.bfloat16)
        cache_ref[p, j] = xb
        n_panels = xb.shape[1] // d
        acc = acc_ref[...]
        cs = cs_ref[...]
        for k in range(n_panels):
            panel = xb[:, k * d:(k + 1) * d]
            acc += jax.lax.dot_general(
                panel, panel, (((0,), (0,)), ((), ())),
                preferred_element_type=jnp.float32)
            cs += jnp.sum(panel.astype(jnp.float32), axis=0, keepdims=True)
        acc_ref[...] = acc
        cs_ref[...] = cs

        @pl.when(j == nc - 1)
        def _ns_epilogue():
            s_raw = acc_ref[...]
            mu = cs_ref[...] * (1.0 / n_samples)
            outer = jax.lax.dot_general(
                mu, mu, (((0,), (0,)), ((), ())),
                preferred_element_type=jnp.float32)
            rows = jax.lax.broadcasted_iota(jnp.int32, (d, d), 0)
            cols = jax.lax.broadcasted_iota(jnp.int32, (d, d), 1)
            eye = jnp.where(rows == cols, jnp.float32(1.0), jnp.float32(0.0))
            s = s_raw - n_samples * outer + _EPS * eye
            norm = jnp.sqrt(jnp.sum(s * s))
            s = s * (1.0 / norm)
            b = 1.5 * eye - 0.5 * s
            s_b = s.astype(jnp.bfloat16)
            for _ in range(_T - 1):
                b_b = b.astype(jnp.bfloat16)
                b2 = jnp.dot(b_b, b_b, preferred_element_type=jnp.float32)
                b3 = jnp.dot(b2.astype(jnp.bfloat16), b_b,
                             preferred_element_type=jnp.float32)
                b = 1.5 * b - 0.5 * jnp.dot(
                    b3.astype(jnp.bfloat16), s_b,
                    preferred_element_type=jnp.float32)
            m = b.T * jax.lax.rsqrt(norm)
            m_ref[p] = m.astype(jnp.bfloat16)
            off_ref[p] = jnp.dot(mu, m, preferred_element_type=jnp.float32)

    @pl.when(i > 0)
    def _apply_phase():
        q = 1 - p
        xb = cache_ref[q, j]
        n_panels = xb.shape[1] // d
        m = m_ref[q]
        off = off_ref[q]
        for k in range(n_panels):
            panel = xb[:, k * d:(k + 1) * d]
            o_ref[:, k * d:(k + 1) * d] = jnp.dot(
                panel, m, preferred_element_type=jnp.float32) - off


def kernel(inputs):
    rows, c = inputs.shape
    g = _G
    d = c // g
    band = rows // g
    n_samples = band * (c // d)

    r = 512
    nc = band // r

    out = pl.pallas_call(
        functools.partial(_oni_kernel, g=g, nc=nc, d=d, n_samples=n_samples),
        grid=(g + 1, nc),
        in_specs=[
            pl.BlockSpec(
                (r, c),
                lambda i, j: (jnp.where(i < g, i * nc + j, g * nc - 1), 0)),
        ],
        out_specs=pl.BlockSpec(
            (r, c),
            lambda i, j: (jnp.where(i > 0, (i - 1) * nc + j, 0), 0)),
        out_shape=jax.ShapeDtypeStruct(inputs.shape, jnp.float32),
        scratch_shapes=[
            pltpu.VMEM((2, nc, r, c), jnp.bfloat16),
            pltpu.VMEM((d, d), jnp.float32),
            pltpu.VMEM((1, d), jnp.float32),
            pltpu.VMEM((2, d, d), jnp.bfloat16),
            pltpu.VMEM((2, 1, d), jnp.float32),
        ],
        compiler_params=pltpu.CompilerParams(
            dimension_semantics=("arbitrary", "arbitrary"),
            vmem_limit_bytes=60 * 1024 * 1024),
        name="oni_fused",
    )(inputs)

    return out
